# baseline (speedup 1.0000x reference)
import functools

import jax
import jax.numpy as jnp
from jax.experimental import pallas as pl
from jax.experimental.pallas import tpu as pltpu

_NUM_HEADS = 16
_SEQ = 2048
_HEAD_DIM = 64
_SCALE = 1.4426950408889634 / (_HEAD_DIM ** 0.5)
_Q_CHUNK = 512


def _attn_kernel(q_ref, k_ref, v_ref, o_ref):
    qs, ks, vaugs = [], [], []
    for h in range(2):
        lo = h * _HEAD_DIM
        qf = q_ref[0, :, lo:lo + _HEAD_DIM]
        kf = k_ref[0, :, lo:lo + _HEAD_DIM]
        qs.append((qf * _SCALE).astype(jnp.bfloat16))
        ks.append(kf.astype(jnp.bfloat16))
        vh = v_ref[0, :, lo:lo + _HEAD_DIM].astype(jnp.bfloat16)
        vaugs.append(jnp.concatenate(
            [vh, jnp.ones((_SEQ, 8), dtype=jnp.bfloat16)], axis=1))
    o_chunks = [[], []]
    for c in range(_SEQ // _Q_CHUNK):
        for h in range(2):
            qc = qs[h][c * _Q_CHUNK:(c + 1) * _Q_CHUNK, :]
            s = jax.lax.dot_general(
                qc, ks[h], (((1,), (1,)), ((), ())),
                preferred_element_type=jnp.float32
            ).astype(jnp.bfloat16)
            m = jnp.max(s, axis=1, keepdims=True)
            p = jnp.exp2(s - m)
            o2 = jax.lax.dot_general(
                p, vaugs[h], (((1,), (0,)), ((), ())),
                preferred_element_type=jnp.float32)
            o_chunks[h].append(
                o2[:, :_HEAD_DIM] / o2[:, _HEAD_DIM:_HEAD_DIM + 1])
    outs = [jnp.concatenate(ch, axis=0) for ch in o_chunks]
    o_ref[0] = jnp.concatenate(outs, axis=1)


@jax.jit
def kernel(q, k, v):
    b, s, dm = q.shape
    grid = (_NUM_HEADS // 2,)
    spec = pl.BlockSpec((1, _SEQ, 2 * _HEAD_DIM), lambda h: (0, 0, h))
    out = pl.pallas_call(
        _attn_kernel,
        grid=grid,
        in_specs=[spec, spec, spec],
        out_specs=spec,
        out_shape=jax.ShapeDtypeStruct((b, s, dm), q.dtype),
        compiler_params=pltpu.CompilerParams(
            dimension_semantics=("parallel",)),
    )(q, k, v)
    return out


# speedup vs baseline: 1.0912x; 1.0912x over previous
import functools

import jax
import jax.numpy as jnp
from jax.experimental import pallas as pl
from jax.experimental.pallas import tpu as pltpu

_NUM_HEADS = 16
_SEQ = 2048
_HEAD_DIM = 64
_SCALE = 1.4426950408889634 / (_HEAD_DIM ** 0.5)
_Q_CHUNK = 1024


def _attn_kernel(q_ref, k_ref, v_ref, o_ref):
    qslab = (q_ref[0] * _SCALE).astype(jnp.bfloat16)
    vaugs = []
    for h in range(2):
        lo = h * _HEAD_DIM
        vh = v_ref[0, :, lo:lo + _HEAD_DIM].astype(jnp.bfloat16)
        vaugs.append(jnp.concatenate(
            [vh, jnp.ones((_SEQ, 8), dtype=jnp.bfloat16)], axis=1))
    kslab = k_ref[0].astype(jnp.bfloat16)
    lane = jax.lax.broadcasted_iota(jnp.int32, (_Q_CHUNK, 2 * _HEAD_DIM), 1)
    zero = jnp.zeros((), dtype=jnp.bfloat16)
    o_chunks = [[], []]
    for c in range(_SEQ // _Q_CHUNK):
        qcs = qslab[c * _Q_CHUNK:(c + 1) * _Q_CHUNK, :]
        qa = jnp.where(lane < _HEAD_DIM, qcs, zero)
        qb = jnp.where(lane >= _HEAD_DIM, qcs, zero)
        sstack = jax.lax.dot_general(
            jnp.concatenate([qa, qb], axis=0), kslab,
            (((1,), (1,)), ((), ())),
            preferred_element_type=jnp.float32
        ).astype(jnp.bfloat16)
        for h in range(2):
            s = sstack[h * _Q_CHUNK:(h + 1) * _Q_CHUNK, :]
            m = jnp.max(s, axis=1, keepdims=True)
            p = jnp.exp2(s - m)
            o2 = jax.lax.dot_general(
                p, vaugs[h], (((1,), (0,)), ((), ())),
                preferred_element_type=jnp.float32)
            o_chunks[h].append(
                o2[:, :_HEAD_DIM] / o2[:, _HEAD_DIM:_HEAD_DIM + 1])
    outs = [jnp.concatenate(ch, axis=0) for ch in o_chunks]
    o_ref[0] = jnp.concatenate(outs, axis=1)


@jax.jit
def kernel(q, k, v):
    b, s, dm = q.shape
    grid = (_NUM_HEADS // 2,)
    spec = pl.BlockSpec((1, _SEQ, 2 * _HEAD_DIM), lambda h: (0, 0, h))
    out = pl.pallas_call(
        _attn_kernel,
        grid=grid,
        in_specs=[spec, spec, spec],
        out_specs=spec,
        out_shape=jax.ShapeDtypeStruct((b, s, dm), q.dtype),
    )(q, k, v)
    return out


# speedup vs baseline: 1.0926x; 1.0013x over previous
import jax
import jax.numpy as jnp
from jax.experimental import pallas as pl

_NUM_HEADS = 16
_SEQ = 2048
_HEAD_DIM = 64
_SCALE = 1.4426950408889634 / (_HEAD_DIM ** 0.5)
_Q_CHUNK = 1024


def _attn_kernel(q_ref, k_ref, v_ref, o_ref):
    qslab = (q_ref[0] * _SCALE).astype(jnp.bfloat16)
    vaugs = []
    for h in range(2):
        lo = h * _HEAD_DIM
        vh = v_ref[0, :, lo:lo + _HEAD_DIM].astype(jnp.bfloat16)
        vaugs.append(jnp.concatenate(
            [vh, jnp.ones((_SEQ, 8), dtype=jnp.bfloat16)], axis=1))
    kslab = k_ref[0].astype(jnp.bfloat16)
    lane = jax.lax.broadcasted_iota(jnp.int32, (_Q_CHUNK, 2 * _HEAD_DIM), 1)
    zero = jnp.zeros((), dtype=jnp.bfloat16)
    o_chunks = [[], []]
    for c in range(_SEQ // _Q_CHUNK):
        qcs = qslab[c * _Q_CHUNK:(c + 1) * _Q_CHUNK, :]
        qa = jnp.where(lane < _HEAD_DIM, qcs, zero)
        qb = jnp.where(lane >= _HEAD_DIM, qcs, zero)
        sstack = jax.lax.dot_general(
            jnp.concatenate([qa, qb], axis=0), kslab,
            (((1,), (1,)), ((), ())),
            preferred_element_type=jnp.float32
        ).astype(jnp.bfloat16)
        for h in range(2):
            s = sstack[h * _Q_CHUNK:(h + 1) * _Q_CHUNK, :]
            m = jnp.max(s, axis=1, keepdims=True)
            p = jnp.exp2(s - m)
            o2 = jax.lax.dot_general(
                p, vaugs[h], (((1,), (0,)), ((), ())),
                preferred_element_type=jnp.float32)
            o_chunks[h].append(
                o2[:, :_HEAD_DIM] / o2[:, _HEAD_DIM:_HEAD_DIM + 1])
    outs = [jnp.concatenate(ch, axis=0) for ch in o_chunks]
    o_ref[0] = jnp.concatenate(outs, axis=1)


@jax.jit
def kernel(q, k, v):
    b, s, dm = q.shape
    grid = (_NUM_HEADS // 2,)
    spec = pl.BlockSpec((1, _SEQ, 2 * _HEAD_DIM), lambda h: (0, 0, h))
    out = pl.pallas_call(
        _attn_kernel,
        grid=grid,
        in_specs=[spec, spec, spec],
        out_specs=spec,
        out_shape=jax.ShapeDtypeStruct((b, s, dm), q.dtype),
    )(q, k, v)
    return out
